# baseline (speedup 1.0000x reference)
import functools

import jax
import jax.numpy as jnp
from jax import lax
from jax.experimental import pallas as pl
from jax.experimental.pallas import tpu as pltpu
from jax.experimental.pallas import tpu_sc as plsc

NUM_WORKERS = 32
CHUNK = 1600


def _sc_gather_add(table, idxp, pos):
    n, d = idxp.shape[0], table.shape[1]
    maxlen = pos.shape[0]
    gpr = 128 // d
    q_rows = CHUNK // gpr
    reps = CHUNK // maxlen
    p_per = maxlen // gpr
    per_w = n // NUM_WORKERS
    n_chunks = per_w // CHUNK
    mesh = plsc.VectorSubcoreMesh(core_axis_name="c", subcore_axis_name="s")

    @functools.partial(
        pl.kernel,
        mesh=mesh,
        out_type=jax.ShapeDtypeStruct((n * d // 128, 128), jnp.float32),
        compiler_params=pltpu.CompilerParams(use_tc_tiling_on_sc=False),
        scratch_types=[
            pltpu.VMEM((CHUNK,), jnp.int32),
            pltpu.VMEM((CHUNK,), jnp.int32),
            pltpu.VMEM((CHUNK, d), jnp.float32),
            pltpu.VMEM((CHUNK, d), jnp.float32),
            pltpu.VMEM((maxlen, d), jnp.float32),
            pltpu.SemaphoreType.DMA,
            pltpu.SemaphoreType.DMA,
            pltpu.SemaphoreType.DMA,
            pltpu.SemaphoreType.DMA,
        ],
    )
    def gather_kernel(table_hbm, idx_hbm, pos_hbm, out_hbm,
                      idx0, idx1, rows0, rows1, pos_v, g0, g1, w0, w1):
        wid = lax.axis_index("s") * 2 + lax.axis_index("c")
        base = wid * per_w
        idx_v = (idx0, idx1)
        rows_v = (rows0, rows1)
        gsem = (g0, g1)
        wsem = (w0, w1)

        pltpu.sync_copy(pos_hbm, pos_v)

        def add_pos(b):
            rows = rows_v[b]

            @pl.loop(0, p_per)
            def _(p):
                for g in range(gpr):
                    for h in range(d // 16):
                        pv = pos_v[gpr * p + g, pl.ds(h * 16, 16)]
                        for t in range(reps):
                            plsc.addupdate(
                                rows.at[g * q_rows + p_per * t + p,
                                        pl.ds(h * 16, 16)], pv)

        def start_gather(ci, b):
            off = base + ci * CHUNK
            pltpu.sync_copy(idx_hbm.at[pl.ds(off, CHUNK)], idx_v[b])
            pltpu.async_copy(table_hbm.at[idx_v[b]], rows_v[b], gsem[b])

        def wait_gather(b):
            pltpu.make_async_copy(table_hbm.at[idx_v[b]], rows_v[b],
                                  gsem[b]).wait()

        def start_writeback(ci, b):
            r0 = (base + ci * CHUNK) // gpr
            for g in range(gpr):
                pltpu.async_copy(
                    rows_v[b].at[pl.ds(g * q_rows, q_rows)],
                    out_hbm.at[pl.ds(r0, q_rows), pl.ds(g * d, d)],
                    wsem[b])

        def wait_writeback(ci, b):
            r0 = (base + ci * CHUNK) // gpr
            for g in range(gpr):
                pltpu.make_async_copy(
                    rows_v[b].at[pl.ds(g * q_rows, q_rows)],
                    out_hbm.at[pl.ds(r0, q_rows), pl.ds(g * d, d)],
                    wsem[b]).wait()

        start_gather(0, 0)

        @pl.loop(0, n_chunks, step=2)
        def _(ci):
            for b in range(2):
                cur = ci + b
                nxt = cur + 1

                @pl.when(nxt < n_chunks)
                def _():
                    @pl.when(nxt >= 2)
                    def _():
                        wait_writeback(nxt - 2, 1 - b)
                    start_gather(nxt, 1 - b)

                wait_gather(b)
                add_pos(b)
                start_writeback(cur, b)

        wait_writeback(n_chunks - 2, 0)
        wait_writeback(n_chunks - 1, 1)

    return gather_kernel(table, idxp, pos)


def kernel(x, token_table, pos_table):
    b, maxlen = x.shape
    d = token_table.shape[1]
    gpr = 128 // d
    n = b * maxlen
    xp = (x.reshape(-1).astype(jnp.int32)
          .reshape(n // CHUNK, CHUNK // gpr, gpr)
          .transpose(0, 2, 1)
          .reshape(-1))
    out128 = _sc_gather_add(token_table, xp, pos_table)
    return out128.reshape(b, maxlen, d)


# speedup vs baseline: 2.2802x; 2.2802x over previous
import functools

import jax
import jax.numpy as jnp
from jax import lax
from jax.experimental import pallas as pl
from jax.experimental.pallas import tpu as pltpu
from jax.experimental.pallas import tpu_sc as plsc

NUM_WORKERS = 32
CHUNK = 1600


def _sc_gather_add(table, idx, pos):
    n, d = idx.shape[0], table.shape[1]
    maxlen = pos.shape[0]
    reps = CHUNK // maxlen
    per_w = n // NUM_WORKERS
    n_chunks = per_w // CHUNK
    mesh = plsc.VectorSubcoreMesh(core_axis_name="c", subcore_axis_name="s")

    @functools.partial(
        pl.kernel,
        mesh=mesh,
        out_type=jax.ShapeDtypeStruct((n, 128), jnp.float32),
        compiler_params=pltpu.CompilerParams(use_tc_tiling_on_sc=False),
        scratch_types=[
            pltpu.VMEM((CHUNK,), jnp.int32),
            pltpu.VMEM((CHUNK,), jnp.int32),
            pltpu.VMEM((CHUNK, d), jnp.float32),
            pltpu.VMEM((CHUNK, d), jnp.float32),
            pltpu.VMEM((maxlen, d), jnp.float32),
            pltpu.SemaphoreType.DMA,
            pltpu.SemaphoreType.DMA,
            pltpu.SemaphoreType.DMA,
            pltpu.SemaphoreType.DMA,
        ],
    )
    def gather_kernel(table_hbm, idx_hbm, pos_hbm, out_hbm,
                      idx0, idx1, rows0, rows1, pos_v, g0, g1, w0, w1):
        wid = lax.axis_index("s") * 2 + lax.axis_index("c")
        base = wid * per_w
        idx_v = (idx0, idx1)
        rows_v = (rows0, rows1)
        gsem = (g0, g1)
        wsem = (w0, w1)

        pltpu.sync_copy(pos_hbm, pos_v)

        def add_pos(b):
            rows = rows_v[b]

            @pl.loop(0, maxlen)
            def _(p):
                for h in range(d // 16):
                    pv = pos_v[p, pl.ds(h * 16, 16)]
                    for t in range(reps):
                        plsc.addupdate(
                            rows.at[t * maxlen + p, pl.ds(h * 16, 16)], pv)

        def start_gather(ci, b):
            off = base + ci * CHUNK
            pltpu.sync_copy(idx_hbm.at[pl.ds(off, CHUNK)], idx_v[b])
            pltpu.async_copy(table_hbm.at[idx_v[b]], rows_v[b], gsem[b])

        def wait_gather(b):
            pltpu.make_async_copy(table_hbm.at[idx_v[b]], rows_v[b],
                                  gsem[b]).wait()

        def start_writeback(ci, b):
            off = base + ci * CHUNK
            pltpu.async_copy(
                rows_v[b],
                out_hbm.at[pl.ds(off, CHUNK), pl.ds(0, d)],
                wsem[b])

        def wait_writeback(ci, b):
            off = base + ci * CHUNK
            pltpu.make_async_copy(
                rows_v[b],
                out_hbm.at[pl.ds(off, CHUNK), pl.ds(0, d)],
                wsem[b]).wait()

        start_gather(0, 0)

        @pl.loop(0, n_chunks, step=2)
        def _(ci):
            for b in range(2):
                cur = ci + b
                nxt = cur + 1

                @pl.when(nxt < n_chunks)
                def _():
                    @pl.when(nxt >= 2)
                    def _():
                        wait_writeback(nxt - 2, 1 - b)
                    start_gather(nxt, 1 - b)

                wait_gather(b)
                add_pos(b)
                start_writeback(cur, b)

        wait_writeback(n_chunks - 2, 0)
        wait_writeback(n_chunks - 1, 1)

    return gather_kernel(table, idx, pos)


def kernel(x, token_table, pos_table):
    b, maxlen = x.shape
    d = token_table.shape[1]
    xf = x.reshape(-1).astype(jnp.int32)
    out128 = _sc_gather_add(token_table, xf, pos_table)
    return out128[:, :d].reshape(b, maxlen, d)


# speedup vs baseline: 2.2836x; 1.0015x over previous
import functools

import jax
import jax.numpy as jnp
from jax import lax
from jax.experimental import pallas as pl
from jax.experimental.pallas import tpu as pltpu
from jax.experimental.pallas import tpu_sc as plsc

NUM_WORKERS = 32
CHUNK = 1600


def _sc_gather_add(table, idx, pos):
    maxlen = pos.shape[0]
    d = table.shape[1]
    n = idx.shape[0] * idx.shape[1]
    reps = CHUNK // maxlen
    per_w = n // NUM_WORKERS
    n_chunks = per_w // CHUNK
    mesh = plsc.VectorSubcoreMesh(core_axis_name="c", subcore_axis_name="s")

    @functools.partial(
        pl.kernel,
        mesh=mesh,
        out_type=jax.ShapeDtypeStruct((n, 128), jnp.float32),
        compiler_params=pltpu.CompilerParams(use_tc_tiling_on_sc=False),
        scratch_types=[
            pltpu.VMEM((reps, maxlen), jnp.int32),
            pltpu.VMEM((reps, maxlen), jnp.int32),
            pltpu.VMEM((CHUNK, d), jnp.float32),
            pltpu.VMEM((CHUNK, d), jnp.float32),
            pltpu.VMEM((maxlen, d), jnp.float32),
            pltpu.SemaphoreType.DMA,
            pltpu.SemaphoreType.DMA,
            pltpu.SemaphoreType.DMA,
            pltpu.SemaphoreType.DMA,
        ],
    )
    def gather_kernel(table_hbm, idx_hbm, pos_hbm, out_hbm,
                      idx0, idx1, rows0, rows1, pos_v, g0, g1, w0, w1):
        wid = lax.axis_index("s") * 2 + lax.axis_index("c")
        base = wid * per_w
        idx_v = (idx0, idx1)
        rows_v = (rows0, rows1)
        gsem = (g0, g1)
        wsem = (w0, w1)

        pltpu.sync_copy(pos_hbm, pos_v)

        def add_pos(b):
            rows = rows_v[b]

            @pl.loop(0, maxlen)
            def _(p):
                for h in range(d // 16):
                    pv = pos_v[p, pl.ds(h * 16, 16)]
                    for t in range(reps):
                        plsc.addupdate(
                            rows.at[t * maxlen + p, pl.ds(h * 16, 16)], pv)

        def start_gather(ci, b):
            row0 = (base + ci * CHUNK) // maxlen
            pltpu.sync_copy(idx_hbm.at[pl.ds(row0, reps)], idx_v[b])
            for k in range(reps):
                pltpu.async_copy(table_hbm.at[idx_v[b].at[k]],
                                 rows_v[b].at[pl.ds(k * maxlen, maxlen)],
                                 gsem[b])

        def wait_gather(b):
            for k in range(reps):
                pltpu.make_async_copy(table_hbm.at[idx_v[b].at[k]],
                                      rows_v[b].at[pl.ds(k * maxlen, maxlen)],
                                      gsem[b]).wait()

        def start_writeback(ci, b):
            off = base + ci * CHUNK
            pltpu.async_copy(
                rows_v[b],
                out_hbm.at[pl.ds(off, CHUNK), pl.ds(0, d)],
                wsem[b])

        def wait_writeback(ci, b):
            off = base + ci * CHUNK
            pltpu.make_async_copy(
                rows_v[b],
                out_hbm.at[pl.ds(off, CHUNK), pl.ds(0, d)],
                wsem[b]).wait()

        start_gather(0, 0)

        @pl.loop(0, n_chunks, step=2)
        def _(ci):
            for b in range(2):
                cur = ci + b
                nxt = cur + 1

                @pl.when(nxt < n_chunks)
                def _():
                    @pl.when(nxt >= 2)
                    def _():
                        wait_writeback(nxt - 2, 1 - b)
                    start_gather(nxt, 1 - b)

                wait_gather(b)
                add_pos(b)
                start_writeback(cur, b)

        wait_writeback(n_chunks - 2, 0)
        wait_writeback(n_chunks - 1, 1)

    return gather_kernel(table, idx, pos)


def kernel(x, token_table, pos_table):
    b, maxlen = x.shape
    d = token_table.shape[1]
    out128 = _sc_gather_add(token_table, x.astype(jnp.int32), pos_table)
    return out128[:, :d].reshape(b, maxlen, d)
